# Initial kernel scaffold; baseline (speedup 1.0000x reference)
import jax
import jax.numpy as jnp
from jax.experimental import pallas as pl


def kernel(feat, edge_index, weight1, weight2):
    raise NotImplementedError("write your pallas kernel here")


# speedup vs baseline: 6.6861x; 6.6861x over previous
import jax
import jax.numpy as jnp
from jax import lax
from jax.experimental import pallas as pl
from jax.experimental.pallas import tpu as pltpu
from jax.experimental.pallas import tpu_sc as plsc

N_N = 10000
N_P = 10240
N_E = 320000
D = 128
NC, NS = 2, 16
NW = NC * NS
B = 80
TPW = N_P // NS


def _hist_body(src_ref, dst_ref, hist_hbm, idx_v, ones_v, zero_v, hist_sh):
    c = lax.axis_index("c")
    s = lax.axis_index("s")
    one = jnp.full((16,), 1.0, jnp.float32)
    zero = jnp.zeros((16,), jnp.float32)
    for k in range(B // 16):
        ones_v[pl.ds(k * 16, 16)] = one
    for k in range(TPW // 16):
        zero_v[pl.ds(k * 16, 16)] = zero
    pltpu.sync_copy(zero_v, hist_sh.at[pl.ds(s * TPW, TPW)])
    plsc.subcore_barrier()
    per = N_E // NS
    base = s * per

    def count(ref):
        def body(i, carry):
            pltpu.sync_copy(ref.at[pl.ds(base + i * B, B)], idx_v)
            pltpu.sync_copy(ones_v, hist_sh.at[idx_v], add=True)
            return carry

        lax.fori_loop(0, per // B, body, 0)

    @pl.when(c == 0)
    def _():
        count(src_ref)

    @pl.when(c == 1)
    def _():
        count(dst_ref)

    plsc.subcore_barrier()
    pltpu.sync_copy(hist_sh.at[pl.ds(s * TPW, TPW)],
                    hist_hbm.at[c, 0, pl.ds(s * TPW, TPW)])


def _agg_body(h_ref, src_ref, dst_ref, part_hbm, sidx, didx, rows, acc_sh, sem):
    c = lax.axis_index("c")
    s = lax.axis_index("s")
    zero = jnp.zeros((16,), jnp.float32)

    def zbody(j, carry):
        rows[j // (D // 16), pl.ds((j % (D // 16)) * 16, 16)] = zero
        return carry

    lax.fori_loop(0, B * (D // 16), zbody, 0)
    for k in range(TPW // B):
        pltpu.sync_copy(rows, acc_sh.at[pl.ds(s * TPW + k * B, B)])
    plsc.subcore_barrier()

    wid = s * NC + c
    base = wid * (N_E // NW)

    def body(i, carry):
        pltpu.sync_copy(src_ref.at[pl.ds(base + i * B, B)], sidx)
        pltpu.sync_copy(dst_ref.at[pl.ds(base + i * B, B)], didx)
        pltpu.async_copy(h_ref.at[sidx], rows, sem).wait()
        pltpu.sync_copy(rows, acc_sh.at[didx], add=True)
        return carry

    lax.fori_loop(0, N_E // NW // B, body, 0)
    plsc.subcore_barrier()
    pltpu.sync_copy(acc_sh.at[pl.ds(s * TPW, TPW)],
                    part_hbm.at[c, pl.ds(s * TPW, TPW)])


def _prep_body(feat_ref, od_ref, h_ref):
    h_ref[...] = feat_ref[...] * lax.rsqrt(jnp.maximum(od_ref[...], 1.0))


def _final_body(p0_ref, p1_ref, h_ref, id_ref, w1_ref, w2_ref, o_ref):
    cs = p0_ref[...] + p1_ref[...]
    nd = lax.rsqrt(jnp.maximum(id_ref[...], 1.0))
    acc = jnp.dot(cs, w1_ref[...], preferred_element_type=jnp.float32)
    acc = acc + jnp.dot(h_ref[...] * cs, w2_ref[...],
                        preferred_element_type=jnp.float32)
    o_ref[...] = acc * nd


_mesh = plsc.VectorSubcoreMesh(core_axis_name="c", subcore_axis_name="s")

_hist = pl.kernel(
    _hist_body,
    out_type=jax.ShapeDtypeStruct((2, 1, N_P), jnp.float32),
    mesh=_mesh,
    scratch_types=[
        pltpu.VMEM((B,), jnp.int32),
        pltpu.VMEM((B,), jnp.float32),
        pltpu.VMEM((TPW,), jnp.float32),
        pltpu.VMEM_SHARED((N_P,), jnp.float32),
    ],
)

_agg = pl.kernel(
    _agg_body,
    out_type=jax.ShapeDtypeStruct((2, N_P, D), jnp.float32),
    mesh=_mesh,
    scratch_types=[
        pltpu.VMEM((B,), jnp.int32),
        pltpu.VMEM((B,), jnp.int32),
        pltpu.VMEM((B, D), jnp.float32),
        pltpu.VMEM_SHARED((N_P, D), jnp.float32),
        pltpu.SemaphoreType.DMA,
    ],
)

_RB = 1000


@jax.jit
def kernel(feat, edge_index, weight1, weight2):
    src = edge_index[0]
    dst = edge_index[1]
    hist = _hist(src, dst)
    od = hist[0, 0, :N_N].reshape(N_N, 1)
    ind = hist[1, 0, :N_N].reshape(N_N, 1)

    h = pl.pallas_call(
        _prep_body,
        grid=(N_N // _RB,),
        in_specs=[
            pl.BlockSpec((_RB, D), lambda i: (i, 0)),
            pl.BlockSpec((_RB, 1), lambda i: (i, 0)),
        ],
        out_specs=pl.BlockSpec((_RB, D), lambda i: (i, 0)),
        out_shape=jax.ShapeDtypeStruct((N_N, D), jnp.float32),
    )(feat, od)

    part = _agg(h, src, dst)

    out = pl.pallas_call(
        _final_body,
        grid=(N_N // _RB,),
        in_specs=[
            pl.BlockSpec((_RB, D), lambda i: (i, 0)),
            pl.BlockSpec((_RB, D), lambda i: (i, 0)),
            pl.BlockSpec((_RB, D), lambda i: (i, 0)),
            pl.BlockSpec((_RB, 1), lambda i: (i, 0)),
            pl.BlockSpec((D, D), lambda i: (0, 0)),
            pl.BlockSpec((D, D), lambda i: (0, 0)),
        ],
        out_specs=pl.BlockSpec((_RB, D), lambda i: (i, 0)),
        out_shape=jax.ShapeDtypeStruct((N_N, D), jnp.float32),
    )(part[0, :N_N], part[1, :N_N], h, ind, weight1, weight2)
    return out


# speedup vs baseline: 16.1784x; 2.4197x over previous
import jax
import jax.numpy as jnp
from jax import lax
from jax.experimental import pallas as pl
from jax.experimental.pallas import tpu as pltpu
from jax.experimental.pallas import tpu_sc as plsc

N_N = 10000
N_P = 10240
N_E = 320000
D = 128
NC, NS = 2, 16
NW = NC * NS
B = 80
TPW = N_P // NS


NB_H = N_E // NS // B
GRP = 10


def _hist_body(src_ref, dst_ref, hist_hbm, idx_all, ones_v, zero_v, hist_sh,
               sem):
    c = lax.axis_index("c")
    s = lax.axis_index("s")
    one = jnp.full((16,), 1.0, jnp.float32)
    zero = jnp.zeros((16,), jnp.float32)
    for k in range(B // 16):
        ones_v[pl.ds(k * 16, 16)] = one
    for k in range(TPW // 16):
        zero_v[pl.ds(k * 16, 16)] = zero
    pltpu.sync_copy(zero_v, hist_sh.at[pl.ds(s * TPW, TPW)])

    base = s * (N_E // NS)

    def fill(ref):
        def fbody(g, carry):
            for k in range(GRP):
                j = g * GRP + k
                pltpu.async_copy(ref.at[pl.ds(base + j * B, B)],
                                 idx_all.at[j], sem)
            for k in range(GRP):
                pltpu.make_async_copy(ref.at[pl.ds(base + k * B, B)],
                                      idx_all.at[k], sem).wait()
            return carry

        lax.fori_loop(0, NB_H // GRP, fbody, 0)

    @pl.when(c == 0)
    def _():
        fill(src_ref)

    @pl.when(c == 1)
    def _():
        fill(dst_ref)

    plsc.subcore_barrier()

    def body(g, carry):
        for k in range(GRP):
            pltpu.async_copy(ones_v, hist_sh.at[idx_all.at[g * GRP + k]], sem,
                             add=True)
        for k in range(GRP):
            pltpu.make_async_copy(ones_v, hist_sh.at[idx_all.at[g * GRP + k]],
                                  sem).wait()
        return carry

    lax.fori_loop(0, NB_H // GRP, body, 0)
    plsc.subcore_barrier()
    pltpu.sync_copy(hist_sh.at[pl.ds(s * TPW, TPW)],
                    hist_hbm.at[c, 0, pl.ds(s * TPW, TPW)])


NB_A = N_E // NW // B


def _agg_body(h_ref, src_ref, dst_ref, part_hbm, sidx_all, didx_all,
              rows0, rows1, acc_sh, gsem0, gsem1, ssem0, ssem1):
    c = lax.axis_index("c")
    s = lax.axis_index("s")
    wid = s * NC + c
    zero = jnp.zeros((16,), jnp.float32)

    def zbody(j, carry):
        for k in range(D // 16):
            rows0[j, pl.ds(k * 16, 16)] = zero
        return carry

    lax.fori_loop(0, B, zbody, 0)
    for k in range(TPW // B):
        pltpu.sync_copy(rows0, acc_sh.at[pl.ds(s * TPW + k * B, B)])
    base = wid * (N_E // NW)
    pltpu.sync_copy(src_ref.at[pl.ds(base, NB_A * B)], sidx_all)

    GRP_F = 25
    def fbody(g, carry):
        for k in range(GRP_F):
            j = g * GRP_F + k
            pltpu.async_copy(dst_ref.at[pl.ds(base + j * B, B)],
                             didx_all.at[j], gsem0)
        for k in range(GRP_F):
            pltpu.make_async_copy(dst_ref.at[pl.ds(base + k * B, B)],
                                  didx_all.at[k], gsem0).wait()
        return carry

    lax.fori_loop(0, NB_A // GRP_F, fbody, 0)
    plsc.subcore_barrier()

    rows = (rows0, rows1)
    gsem = (gsem0, gsem1)
    ssem = (ssem0, ssem1)

    pltpu.async_copy(h_ref.at[sidx_all.at[pl.ds(0, B)]], rows0, gsem0)
    pltpu.async_copy(h_ref.at[sidx_all.at[pl.ds(B, B)]], rows1, gsem1)

    def body(i, carry):
        for b in range(2):
            j = 2 * i + b

            @pl.when(j < NB_A)
            def _():
                pltpu.make_async_copy(h_ref.at[sidx_all.at[pl.ds(j * B, B)]],
                                      rows[b], gsem[b]).wait()
                pltpu.async_copy(rows[b], acc_sh.at[didx_all.at[j]], ssem[b],
                                 add=True)
                pltpu.make_async_copy(rows[b], acc_sh.at[didx_all.at[j]],
                                      ssem[b]).wait()

                @pl.when(j + 2 < NB_A)
                def _():
                    pltpu.async_copy(
                        h_ref.at[sidx_all.at[pl.ds((j + 2) * B, B)]],
                        rows[b], gsem[b])

        return carry

    lax.fori_loop(0, (NB_A + 2) // 2, body, 0)
    plsc.subcore_barrier()
    pltpu.sync_copy(acc_sh.at[pl.ds(s * TPW, TPW)],
                    part_hbm.at[c, pl.ds(s * TPW, TPW)])


def _prep_body(feat_ref, od_ref, h_ref):
    h_ref[...] = feat_ref[...] * lax.rsqrt(jnp.maximum(od_ref[...], 1.0))


def _final_body(p0_ref, p1_ref, h_ref, id_ref, w1_ref, w2_ref, o_ref):
    cs = p0_ref[...] + p1_ref[...]
    nd = lax.rsqrt(jnp.maximum(id_ref[...], 1.0))
    acc = jnp.dot(cs, w1_ref[...], preferred_element_type=jnp.float32)
    acc = acc + jnp.dot(h_ref[...] * cs, w2_ref[...],
                        preferred_element_type=jnp.float32)
    o_ref[...] = acc * nd


_mesh = plsc.VectorSubcoreMesh(core_axis_name="c", subcore_axis_name="s")

_hist = pl.kernel(
    _hist_body,
    out_type=jax.ShapeDtypeStruct((2, 1, N_P), jnp.float32),
    mesh=_mesh,
    scratch_types=[
        pltpu.VMEM((NB_H, B), jnp.int32),
        pltpu.VMEM((B,), jnp.float32),
        pltpu.VMEM((TPW,), jnp.float32),
        pltpu.VMEM_SHARED((N_P,), jnp.float32),
        pltpu.SemaphoreType.DMA,
    ],
)

_agg = pl.kernel(
    _agg_body,
    out_type=jax.ShapeDtypeStruct((2, N_P, D), jnp.float32),
    mesh=_mesh,
    scratch_types=[
        pltpu.VMEM((NB_A * B,), jnp.int32),
        pltpu.VMEM((NB_A, B), jnp.int32),
        pltpu.VMEM((B, D), jnp.float32),
        pltpu.VMEM((B, D), jnp.float32),
        pltpu.VMEM_SHARED((N_P, D), jnp.float32),
        pltpu.SemaphoreType.DMA,
        pltpu.SemaphoreType.DMA,
        pltpu.SemaphoreType.DMA,
        pltpu.SemaphoreType.DMA,
    ],
)

_RB = 1000


@jax.jit
def kernel(feat, edge_index, weight1, weight2):
    src = edge_index[0]
    dst = edge_index[1]
    hist = _hist(src, dst)
    od = hist[0, 0, :N_N].reshape(N_N, 1)
    ind = hist[1, 0, :N_N].reshape(N_N, 1)

    h = pl.pallas_call(
        _prep_body,
        grid=(N_N // _RB,),
        in_specs=[
            pl.BlockSpec((_RB, D), lambda i: (i, 0)),
            pl.BlockSpec((_RB, 1), lambda i: (i, 0)),
        ],
        out_specs=pl.BlockSpec((_RB, D), lambda i: (i, 0)),
        out_shape=jax.ShapeDtypeStruct((N_N, D), jnp.float32),
    )(feat, od)

    part = _agg(h, src, dst)

    out = pl.pallas_call(
        _final_body,
        grid=(N_N // _RB,),
        in_specs=[
            pl.BlockSpec((_RB, D), lambda i: (i, 0)),
            pl.BlockSpec((_RB, D), lambda i: (i, 0)),
            pl.BlockSpec((_RB, D), lambda i: (i, 0)),
            pl.BlockSpec((_RB, 1), lambda i: (i, 0)),
            pl.BlockSpec((D, D), lambda i: (0, 0)),
            pl.BlockSpec((D, D), lambda i: (0, 0)),
        ],
        out_specs=pl.BlockSpec((_RB, D), lambda i: (i, 0)),
        out_shape=jax.ShapeDtypeStruct((N_N, D), jnp.float32),
    )(part[0, :N_N], part[1, :N_N], h, ind, weight1, weight2)
    return out
